# speedup vs baseline: 2.6525x; 2.6525x over previous
import functools

import jax
import jax.numpy as jnp
import numpy as np
from jax.experimental import pallas as pl
from jax.experimental.pallas import tpu as pltpu

N = 10000
LMAX = 3
NCOMP = (LMAX + 1) ** 2
C = 96
E = 16
EP = 128
PATH_WEIGHT = 1.0 / np.sqrt(C)

BLOCK_N = 1000


def _body(x_ref, y_ref, w_ref, o_ref):
    yv = y_ref[...]
    ycols = [yv[:, e : e + 1] for e in range(E)]
    for l in range(LMAX + 1):
        s = l * l
        wl = w_ref[l]
        for m in range(s, s + 2 * l + 1):
            xm = x_ref[:, m, :]
            t = jnp.dot(xm, wl, preferred_element_type=jnp.float32)
            acc = t[:, 0:C] * ycols[0]
            for e in range(1, E):
                acc = acc + t[:, e * EP : e * EP + C] * ycols[e]
            o_ref[:, m, :] = acc


@jax.jit
def kernel(x, y, W0, W1, W2, W3):
    Ws = jnp.stack([W0, W1, W2, W3])
    Wt = jnp.transpose(Ws, (0, 2, 1, 3)) * PATH_WEIGHT
    Wp = jnp.pad(Wt, ((0, 0), (0, 0), (0, 0), (0, EP - C)))
    Wcat = Wp.reshape(4, C, E * EP).astype(jnp.bfloat16)
    xb = x.astype(jnp.bfloat16)

    grid = (N // BLOCK_N,)
    out = pl.pallas_call(
        _body,
        grid=grid,
        in_specs=[
            pl.BlockSpec((BLOCK_N, NCOMP, C), lambda i: (i, 0, 0)),
            pl.BlockSpec((BLOCK_N, E), lambda i: (i, 0)),
            pl.BlockSpec((4, C, E * EP), lambda i: (0, 0, 0)),
        ],
        out_specs=pl.BlockSpec((BLOCK_N, NCOMP, C), lambda i: (i, 0, 0)),
        out_shape=jax.ShapeDtypeStruct((N, NCOMP, C), jnp.float32),
    )(xb, y, Wcat)
    return out
